# baseline (speedup 1.0000x reference)
import functools

import jax
import jax.numpy as jnp
from jax import lax
from jax.experimental import pallas as pl
from jax.experimental.pallas import tpu as pltpu
from jax.experimental.pallas import tpu_sc as plsc

NC = 2
NS = 16
NW = NC * NS
LANES = 16
CHUNK = 128
IB = 8
HIDP = 16


def _deg_body(dst_hbm, out_a, out_b, idx_v, ones_v, stage_v, deg_sh, ssem):
    n_pad = deg_sh.shape[0]
    rows = n_pad // NS
    n_b = dst_hbm.shape[1] // IB
    cid = lax.axis_index("c")
    sid = lax.axis_index("s")
    wid = sid * NC + cid

    def obody(i, carry):
        ones_v[i] = jnp.ones((LANES,), jnp.float32)
        return carry

    lax.fori_loop(0, CHUNK, obody, 0)

    def zbody(i, carry):
        stage_v[i] = jnp.zeros((LANES,), jnp.float32)
        return carry

    lax.fori_loop(0, rows, zbody, 0)
    pltpu.sync_copy(stage_v, deg_sh.at[pl.ds(sid * rows, rows)])
    plsc.subcore_barrier()

    def outer(bi, carry):
        pltpu.sync_copy(dst_hbm.at[wid, pl.ds(bi * IB, IB)], idx_v)
        descs = [
            pltpu.async_copy(ones_v, deg_sh.at[idx_v.at[j]], ssem, add=True)
            for j in range(IB)
        ]
        for d in descs:
            d.wait()
        return carry

    lax.fori_loop(0, n_b, outer, 0)
    plsc.subcore_barrier()
    pltpu.sync_copy(deg_sh.at[pl.ds(sid * rows, rows)], stage_v)

    @pl.when(cid == 0)
    def _():
        pltpu.sync_copy(stage_v, out_a.at[pl.ds(sid * rows, rows)])

    @pl.when(cid == 1)
    def _():
        pltpu.sync_copy(stage_v, out_b.at[pl.ds(sid * rows, rows)])


def _edge_body(src_hbm, dst_hbm, table_hbm, out_a, out_b,
               sidx_v, didx_v, rows_v, zbuf_v, table_sh, acc_sh,
               gsems, ssem):
    n_pad = acc_sh.shape[0]
    rows = n_pad // NS
    n_b = src_hbm.shape[1] // IB
    cid = lax.axis_index("c")
    sid = lax.axis_index("s")
    wid = sid * NC + cid
    base = sid * rows
    pieces = [(o, min(CHUNK, rows - o)) for o in range(0, rows, CHUNK)]

    def zbody(i, carry):
        zbuf_v[i] = jnp.zeros((LANES,), jnp.float32)
        return carry

    lax.fori_loop(0, CHUNK, zbody, 0)
    for o, ln in pieces:
        pltpu.sync_copy(zbuf_v.at[pl.ds(0, ln)],
                        acc_sh.at[pl.ds(base + o, ln)])
    tds = [
        pltpu.async_copy(table_hbm.at[pl.ds(base + o, ln)],
                         rows_v.at[k, pl.ds(0, ln)], gsems.at[k])
        for k, (o, ln) in enumerate(pieces)
    ]
    for k, (o, ln) in enumerate(pieces):
        tds[k].wait()
        pltpu.sync_copy(rows_v.at[k, pl.ds(0, ln)],
                        table_sh.at[pl.ds(base + o, ln)])
    plsc.subcore_barrier()

    def outer(bi, carry):
        i0 = pltpu.async_copy(src_hbm.at[wid, pl.ds(bi * IB, IB)], sidx_v,
                              ssem)
        i1 = pltpu.async_copy(dst_hbm.at[wid, pl.ds(bi * IB, IB)], didx_v,
                              ssem)
        i0.wait()
        i1.wait()
        gd = [
            pltpu.async_copy(table_sh.at[sidx_v.at[j]], rows_v.at[j],
                             gsems.at[j])
            for j in range(IB)
        ]
        sd = []
        for j in range(IB):
            gd[j].wait()
            sd.append(
                pltpu.async_copy(rows_v.at[j], acc_sh.at[didx_v.at[j]],
                                 ssem, add=True))
        for d in sd:
            d.wait()
        return carry

    lax.fori_loop(0, n_b, outer, 0)
    plsc.subcore_barrier()
    for k, (o, ln) in enumerate(pieces):
        pltpu.sync_copy(acc_sh.at[pl.ds(base + o, ln)],
                        rows_v.at[k, pl.ds(0, ln)])

        @pl.when(cid == 0)
        def _(k=k, o=o, ln=ln):
            pltpu.sync_copy(rows_v.at[k, pl.ds(0, ln)],
                            out_a.at[pl.ds(base + o, ln)])

        @pl.when(cid == 1)
        def _(k=k, o=o, ln=ln):
            pltpu.sync_copy(rows_v.at[k, pl.ds(0, ln)],
                            out_b.at[pl.ds(base + o, ln)])


def _tc0_body(x_ref, w_ref, h_ref):
    h_ref[...] = jnp.dot(x_ref[...], w_ref[...],
                         preferred_element_type=jnp.float32)


def _tc1_body(h_ref, da_ref, db_ref, hn_ref, dinv_ref):
    deg = da_ref[...] + db_ref[...] + 1.0
    dinv = lax.rsqrt(deg)
    hn_ref[...] = h_ref[...] * dinv
    dinv_ref[...] = dinv


def _tc2_body(pa_ref, pb_ref, hn1_ref, dinv_ref, b1_ref, w2_ref, hn2_ref):
    s = pa_ref[...] + pb_ref[...] + hn1_ref[...]
    h1 = jnp.maximum(dinv_ref[...] * s + b1_ref[...], 0.0)
    hn2_ref[...] = (
        jnp.dot(h1, w2_ref[...], preferred_element_type=jnp.float32)
        * dinv_ref[...])


def _tc3_body(pa_ref, pb_ref, hn2_ref, dinv_ref, b2_ref, out_ref):
    s = pa_ref[...] + pb_ref[...] + hn2_ref[...]
    out_ref[...] = dinv_ref[...] * s + b2_ref[...]


@functools.partial(jax.jit, static_argnames=("n_pad", "cpw"))
def _run(x_p, srcp, dstp, W1, b1r, w2p, b2r, n_pad, cpw):
    f32 = jnp.float32
    mesh = plsc.VectorSubcoreMesh(core_axis_name="c", subcore_axis_name="s")
    rows = n_pad // NS

    sc_params = pltpu.CompilerParams(use_tc_tiling_on_sc=False)
    sds = jax.ShapeDtypeStruct((n_pad, HIDP), f32)
    deg_call = pl.kernel(
        _deg_body,
        out_type=[sds, sds],
        mesh=mesh,
        compiler_params=sc_params,
        scratch_types=[
            pltpu.VMEM((IB, CHUNK), jnp.int32),
            pltpu.VMEM((CHUNK, HIDP), f32),
            pltpu.VMEM((rows, HIDP), f32),
            pltpu.VMEM_SHARED((n_pad, HIDP), f32),
            pltpu.SemaphoreType.DMA,
        ],
    )
    edge_call = pl.kernel(
        _edge_body,
        out_type=[sds, sds],
        mesh=mesh,
        compiler_params=sc_params,
        scratch_types=[
            pltpu.VMEM((IB, CHUNK), jnp.int32),
            pltpu.VMEM((IB, CHUNK), jnp.int32),
            pltpu.VMEM((IB, CHUNK, HIDP), f32),
            pltpu.VMEM((CHUNK, HIDP), f32),
            pltpu.VMEM_SHARED((n_pad, HIDP), f32),
            pltpu.VMEM_SHARED((n_pad, HIDP), f32),
            pltpu.SemaphoreType.DMA((IB,)),
            pltpu.SemaphoreType.DMA,
        ],
    )

    grid = 8
    blk = n_pad // grid
    row_spec = pl.BlockSpec((blk, HIDP), lambda i: (i, 0))
    one_spec = pl.BlockSpec((1, HIDP), lambda i: (0, 0))

    da, db = deg_call(dstp)
    h_raw = pl.pallas_call(
        _tc0_body,
        grid=(grid,),
        in_specs=[pl.BlockSpec((blk, x_p.shape[1]), lambda i: (i, 0)),
                  pl.BlockSpec((x_p.shape[1], HIDP), lambda i: (0, 0))],
        out_specs=row_spec,
        out_shape=sds,
    )(x_p, W1)
    hn1, dinv = pl.pallas_call(
        _tc1_body,
        grid=(grid,),
        in_specs=[row_spec, row_spec, row_spec],
        out_specs=(row_spec, row_spec),
        out_shape=(sds, sds),
    )(h_raw, da, db)
    p1a, p1b = edge_call(srcp, dstp, hn1)
    hn2 = pl.pallas_call(
        _tc2_body,
        grid=(grid,),
        in_specs=[row_spec, row_spec, row_spec, row_spec, one_spec,
                  pl.BlockSpec((HIDP, HIDP), lambda i: (0, 0))],
        out_specs=row_spec,
        out_shape=sds,
    )(p1a, p1b, hn1, dinv, b1r, w2p)
    p2a, p2b = edge_call(srcp, dstp, hn2)
    outf = pl.pallas_call(
        _tc3_body,
        grid=(grid,),
        in_specs=[row_spec, row_spec, row_spec, row_spec, one_spec],
        out_specs=row_spec,
        out_shape=sds,
    )(p2a, p2b, hn2, dinv, b2r)
    return outf


def kernel(x, edge_index, W1, b1, W2, b2):
    f32 = jnp.float32
    n, in_ch = x.shape
    hid = W1.shape[1]
    out_ch = W2.shape[1]
    n_edges = edge_index.shape[1]

    block = NW * CHUNK * IB
    cpw = IB * (-(-n_edges // block))
    e_pad = cpw * NW * CHUNK
    n_pad = -(-n // (NS * 8)) * (NS * 8)
    dummy = n

    ei = edge_index.astype(jnp.int32)
    pad = jnp.full((e_pad - n_edges,), dummy, jnp.int32)
    srcp = jnp.concatenate([ei[0], pad]).reshape(NW, cpw, CHUNK)
    dstp = jnp.concatenate([ei[1], pad]).reshape(NW, cpw, CHUNK)
    x_p = jnp.pad(x.astype(f32), ((0, n_pad - n), (0, 0)))
    b1r = b1.reshape(1, hid).astype(f32)
    w2p = jnp.pad(W2.astype(f32), ((0, 0), (0, HIDP - out_ch)))
    b2r = jnp.pad(b2.reshape(1, out_ch).astype(f32),
                  ((0, 0), (0, HIDP - out_ch)))

    outf = _run(x_p, srcp, dstp, W1.astype(f32), b1r, w2p, b2r,
                n_pad=n_pad, cpw=cpw)
    return outf[:n, :out_ch]


# speedup vs baseline: 1.1425x; 1.1425x over previous
import functools

import jax
import jax.numpy as jnp
from jax import lax
from jax.experimental import pallas as pl
from jax.experimental.pallas import tpu as pltpu
from jax.experimental.pallas import tpu_sc as plsc

NC = 2
NS = 16
NW = NC * NS
LANES = 16
CHUNK = 128
IB = 8
HIDP = 16


def _rsqrt_newton(x):
    i = plsc.bitcast(x, jnp.int32)
    y = plsc.bitcast(jnp.int32(0x5F3759DF) - (i >> 1), jnp.float32)
    for _ in range(4):
        y = y * (1.5 - 0.5 * x * y * y)
    return y


def _deg_body(dst_hbm, out_a, out_b, idx_v, ones_v, stage_v, deg_sh, ssem):
    n_pad = deg_sh.shape[0]
    rows = n_pad // NS
    n_b = dst_hbm.shape[1] // IB
    cid = lax.axis_index("c")
    sid = lax.axis_index("s")
    wid = sid * NC + cid

    def obody(i, carry):
        ones_v[i] = jnp.ones((LANES,), jnp.float32)
        return carry

    lax.fori_loop(0, CHUNK, obody, 0)

    def zbody(i, carry):
        stage_v[i] = jnp.zeros((LANES,), jnp.float32)
        return carry

    lax.fori_loop(0, rows, zbody, 0)
    pltpu.sync_copy(stage_v, deg_sh.at[pl.ds(sid * rows, rows)])
    plsc.subcore_barrier()

    def outer(bi, carry):
        pltpu.sync_copy(dst_hbm.at[wid, pl.ds(bi * IB, IB)], idx_v)
        descs = [
            pltpu.async_copy(ones_v, deg_sh.at[idx_v.at[j]], ssem, add=True)
            for j in range(IB)
        ]
        for d in descs:
            d.wait()
        return carry

    lax.fori_loop(0, n_b, outer, 0)
    plsc.subcore_barrier()
    pltpu.sync_copy(deg_sh.at[pl.ds(sid * rows, rows)], stage_v)

    @pl.when(cid == 0)
    def _():
        pltpu.sync_copy(stage_v, out_a.at[pl.ds(sid * rows, rows)])

    @pl.when(cid == 1)
    def _():
        pltpu.sync_copy(stage_v, out_b.at[pl.ds(sid * rows, rows)])


def _zero_acc(zbuf_v, acc_sh, base, pieces):
    def zbody(i, carry):
        zbuf_v[i] = jnp.zeros((LANES,), jnp.float32)
        return carry

    lax.fori_loop(0, CHUNK, zbody, 0)
    for o, ln in pieces:
        pltpu.sync_copy(zbuf_v.at[pl.ds(0, ln)],
                        acc_sh.at[pl.ds(base + o, ln)])


def _edge_loop(src_hbm, dst_hbm, sidx_v, didx_v, rows_v, table_sh, acc_sh,
               gsems, ssem, wid):
    n_b = src_hbm.shape[1] // IB

    def outer(bi, carry):
        i0 = pltpu.async_copy(src_hbm.at[wid, pl.ds(bi * IB, IB)], sidx_v,
                              ssem)
        i1 = pltpu.async_copy(dst_hbm.at[wid, pl.ds(bi * IB, IB)], didx_v,
                              ssem)
        i0.wait()
        i1.wait()
        gd = [
            pltpu.async_copy(table_sh.at[sidx_v.at[j]],
                             rows_v.at[pl.ds(j * CHUNK, CHUNK)], gsems.at[j])
            for j in range(IB)
        ]
        sd = []
        for j in range(IB):
            gd[j].wait()
            sd.append(
                pltpu.async_copy(rows_v.at[pl.ds(j * CHUNK, CHUNK)],
                                 acc_sh.at[didx_v.at[j]], ssem, add=True))
        for d in sd:
            d.wait()
        return carry

    lax.fori_loop(0, n_b, outer, 0)


def _acc_writeback(rows_v, acc_sh, out_a, out_b, base, pieces, cid):
    for k, (o, ln) in enumerate(pieces):
        pltpu.sync_copy(acc_sh.at[pl.ds(base + o, ln)],
                        rows_v.at[pl.ds(k * CHUNK, ln)])

        @pl.when(cid == 0)
        def _(k=k, o=o, ln=ln):
            pltpu.sync_copy(rows_v.at[pl.ds(k * CHUNK, ln)],
                            out_a.at[pl.ds(base + o, ln)])

        @pl.when(cid == 1)
        def _(k=k, o=o, ln=ln):
            pltpu.sync_copy(rows_v.at[pl.ds(k * CHUNK, ln)],
                            out_b.at[pl.ds(base + o, ln)])


def _edge1_body(src_hbm, dst_hbm, da_hbm, db_hbm, h_hbm,
                pa_out, pb_out, hn1_out, dinv_out,
                sidx_v, didx_v, rows_v, zbuf_v, table_sh, acc_sh,
                gsems, ssem):
    n_pad = acc_sh.shape[0]
    rows = n_pad // NS
    cid = lax.axis_index("c")
    sid = lax.axis_index("s")
    wid = sid * NC + cid
    base = sid * rows
    pieces = [(o, min(CHUNK, rows - o)) for o in range(0, rows, CHUNK)]

    _zero_acc(zbuf_v, acc_sh, base, pieces)
    for o, ln in pieces:
        c0 = pltpu.async_copy(da_hbm.at[pl.ds(base + o, ln)],
                              rows_v.at[pl.ds(0, ln)], gsems.at[0])
        c1 = pltpu.async_copy(db_hbm.at[pl.ds(base + o, ln)],
                              rows_v.at[pl.ds(CHUNK, ln)], gsems.at[1])
        c2 = pltpu.async_copy(h_hbm.at[pl.ds(base + o, ln)],
                              rows_v.at[pl.ds(2 * CHUNK, ln)], gsems.at[2])
        c0.wait()
        c1.wait()
        c2.wait()

        def body(i, carry):
            d = rows_v[i] + rows_v[CHUNK + i] + 1.0
            y = _rsqrt_newton(d)
            rows_v[3 * CHUNK + i] = y
            rows_v[4 * CHUNK + i] = rows_v[2 * CHUNK + i] * y
            return carry

        lax.fori_loop(0, ln, body, 0)
        pltpu.sync_copy(rows_v.at[pl.ds(4 * CHUNK, ln)],
                        table_sh.at[pl.ds(base + o, ln)])

        @pl.when(cid == 0)
        def _(o=o, ln=ln):
            pltpu.sync_copy(rows_v.at[pl.ds(4 * CHUNK, ln)],
                            hn1_out.at[pl.ds(base + o, ln)])
            pltpu.sync_copy(rows_v.at[pl.ds(3 * CHUNK, ln)],
                            dinv_out.at[pl.ds(base + o, ln)])

    plsc.subcore_barrier()
    _edge_loop(src_hbm, dst_hbm, sidx_v, didx_v, rows_v, table_sh, acc_sh,
               gsems, ssem, wid)
    plsc.subcore_barrier()
    _acc_writeback(rows_v, acc_sh, pa_out, pb_out, base, pieces, cid)


def _edge2_body(src_hbm, dst_hbm, pa_hbm, pb_hbm, hn1_hbm, dinv_hbm, b1_hbm,
                pa_out, pb_out, r_out,
                sidx_v, didx_v, rows_v, zbuf_v, b1_v, table_sh, acc_sh,
                gsems, ssem):
    n_pad = acc_sh.shape[0]
    rows = n_pad // NS
    cid = lax.axis_index("c")
    sid = lax.axis_index("s")
    wid = sid * NC + cid
    base = sid * rows
    pieces = [(o, min(CHUNK, rows - o)) for o in range(0, rows, CHUNK)]

    _zero_acc(zbuf_v, acc_sh, base, pieces)
    pltpu.sync_copy(b1_hbm, b1_v)
    b1 = b1_v[...]
    for o, ln in pieces:
        c0 = pltpu.async_copy(pa_hbm.at[pl.ds(base + o, ln)],
                              rows_v.at[pl.ds(0, ln)], gsems.at[0])
        c1 = pltpu.async_copy(pb_hbm.at[pl.ds(base + o, ln)],
                              rows_v.at[pl.ds(CHUNK, ln)], gsems.at[1])
        c2 = pltpu.async_copy(hn1_hbm.at[pl.ds(base + o, ln)],
                              rows_v.at[pl.ds(2 * CHUNK, ln)], gsems.at[2])
        c3 = pltpu.async_copy(dinv_hbm.at[pl.ds(base + o, ln)],
                              rows_v.at[pl.ds(3 * CHUNK, ln)], gsems.at[3])
        c0.wait()
        c1.wait()
        c2.wait()
        c3.wait()

        def body(i, carry):
            s = rows_v[i] + rows_v[CHUNK + i] + rows_v[2 * CHUNK + i]
            y = rows_v[3 * CHUNK + i]
            o1 = jnp.maximum(y * s + b1, 0.0)
            rows_v[4 * CHUNK + i] = y * o1
            return carry

        lax.fori_loop(0, ln, body, 0)
        pltpu.sync_copy(rows_v.at[pl.ds(4 * CHUNK, ln)],
                        table_sh.at[pl.ds(base + o, ln)])

        @pl.when(cid == 0)
        def _(o=o, ln=ln):
            pltpu.sync_copy(rows_v.at[pl.ds(4 * CHUNK, ln)],
                            r_out.at[pl.ds(base + o, ln)])

    plsc.subcore_barrier()
    _edge_loop(src_hbm, dst_hbm, sidx_v, didx_v, rows_v, table_sh, acc_sh,
               gsems, ssem, wid)
    plsc.subcore_barrier()
    _acc_writeback(rows_v, acc_sh, pa_out, pb_out, base, pieces, cid)


def _tc0_body(x_ref, w_ref, h_ref):
    h_ref[...] = jnp.dot(x_ref[...], w_ref[...],
                         preferred_element_type=jnp.float32)


def _tcf_body(pa_ref, pb_ref, r_ref, dinv_ref, w2_ref, b2_ref, out_ref):
    t = pa_ref[...] + pb_ref[...] + r_ref[...]
    out_ref[...] = (
        dinv_ref[...]
        * jnp.dot(t, w2_ref[...], preferred_element_type=jnp.float32)
        + b2_ref[...])


@functools.partial(jax.jit, static_argnames=("n_pad", "cpw"))
def _run(x_p, srcp, dstp, W1, b1v, w2p, b2r, n_pad, cpw):
    f32 = jnp.float32
    mesh = plsc.VectorSubcoreMesh(core_axis_name="c", subcore_axis_name="s")
    rows = n_pad // NS

    sc_params = pltpu.CompilerParams(use_tc_tiling_on_sc=False,
                                     needs_layout_passes=False)
    sds = jax.ShapeDtypeStruct((n_pad, HIDP), f32)
    deg_call = pl.kernel(
        _deg_body,
        out_type=[sds, sds],
        mesh=mesh,
        compiler_params=sc_params,
        scratch_types=[
            pltpu.VMEM((IB, CHUNK), jnp.int32),
            pltpu.VMEM((CHUNK, HIDP), f32),
            pltpu.VMEM((rows, HIDP), f32),
            pltpu.VMEM_SHARED((n_pad, HIDP), f32),
            pltpu.SemaphoreType.DMA,
        ],
    )
    edge_scratch = [
        pltpu.VMEM((IB, CHUNK), jnp.int32),
        pltpu.VMEM((IB, CHUNK), jnp.int32),
        pltpu.VMEM((IB * CHUNK, HIDP), f32),
        pltpu.VMEM((CHUNK, HIDP), f32),
        pltpu.VMEM_SHARED((n_pad, HIDP), f32),
        pltpu.VMEM_SHARED((n_pad, HIDP), f32),
        pltpu.SemaphoreType.DMA((IB,)),
        pltpu.SemaphoreType.DMA,
    ]
    edge1_call = pl.kernel(
        _edge1_body,
        out_type=[sds, sds, sds, sds],
        mesh=mesh,
        compiler_params=sc_params,
        scratch_types=edge_scratch,
    )
    edge2_call = pl.kernel(
        _edge2_body,
        out_type=[sds, sds, sds],
        mesh=mesh,
        compiler_params=sc_params,
        scratch_types=(edge_scratch[:4]
                       + [pltpu.VMEM((LANES,), f32)]
                       + edge_scratch[4:]),
    )

    grid = 8
    blk = n_pad // grid
    row_spec = pl.BlockSpec((blk, HIDP), lambda i: (i, 0))
    one_spec = pl.BlockSpec((1, HIDP), lambda i: (0, 0))

    da, db = deg_call(dstp)
    h_raw = pl.pallas_call(
        _tc0_body,
        grid=(grid,),
        in_specs=[pl.BlockSpec((blk, x_p.shape[1]), lambda i: (i, 0)),
                  pl.BlockSpec((x_p.shape[1], HIDP), lambda i: (0, 0))],
        out_specs=row_spec,
        out_shape=sds,
    )(x_p, W1)
    p1a, p1b, hn1, dinv = edge1_call(srcp, dstp, da, db, h_raw)
    p2a, p2b, r = edge2_call(srcp, dstp, p1a, p1b, hn1, dinv, b1v)
    outf = pl.pallas_call(
        _tcf_body,
        grid=(grid,),
        in_specs=[row_spec, row_spec, row_spec, row_spec,
                  pl.BlockSpec((HIDP, HIDP), lambda i: (0, 0)), one_spec],
        out_specs=row_spec,
        out_shape=sds,
    )(p2a, p2b, r, dinv, w2p, b2r)
    return outf


def kernel(x, edge_index, W1, b1, W2, b2):
    f32 = jnp.float32
    n, in_ch = x.shape
    out_ch = W2.shape[1]
    n_edges = edge_index.shape[1]

    block = NW * CHUNK * IB
    cpw = IB * (-(-n_edges // block))
    e_pad = cpw * NW * CHUNK
    n_pad = -(-n // (NS * 8)) * (NS * 8)
    dummy = n

    ei = edge_index.astype(jnp.int32)
    pad = jnp.full((e_pad - n_edges,), dummy, jnp.int32)
    srcp = jnp.concatenate([ei[0], pad]).reshape(NW, cpw, CHUNK)
    dstp = jnp.concatenate([ei[1], pad]).reshape(NW, cpw, CHUNK)
    x_p = jnp.pad(x.astype(f32), ((0, n_pad - n), (0, 0)))
    b1v = b1.astype(f32)
    w2p = jnp.pad(W2.astype(f32), ((0, 0), (0, HIDP - out_ch)))
    b2r = jnp.pad(b2.reshape(1, out_ch).astype(f32),
                  ((0, 0), (0, HIDP - out_ch)))

    outf = _run(x_p, srcp, dstp, W1.astype(f32), b1v, w2p, b2r,
                n_pad=n_pad, cpw=cpw)
    return outf[:n, :out_ch]


# speedup vs baseline: 1.1514x; 1.0078x over previous
import functools

import jax
import jax.numpy as jnp
from jax import lax
from jax.experimental import pallas as pl
from jax.experimental.pallas import tpu as pltpu
from jax.experimental.pallas import tpu_sc as plsc

NC = 2
NS = 16
NW = NC * NS
LANES = 16
CHUNK = 128
IB = 8
HIDP = 16


def _rsqrt_newton(x):
    i = plsc.bitcast(x, jnp.int32)
    y = plsc.bitcast(jnp.int32(0x5F3759DF) - (i >> 1), jnp.float32)
    for _ in range(4):
        y = y * (1.5 - 0.5 * x * y * y)
    return y


def _deg_body(dst_hbm, out_a, out_b, idx_v, ones_v, stage_v, deg_sh, ssem):
    n_pad = deg_sh.shape[0]
    rows = n_pad // NS
    n_b = dst_hbm.shape[1] // IB
    cid = lax.axis_index("c")
    sid = lax.axis_index("s")
    wid = sid * NC + cid

    def obody(i, carry):
        ones_v[i] = jnp.ones((LANES,), jnp.float32)
        return carry

    lax.fori_loop(0, CHUNK, obody, 0)

    def zbody(i, carry):
        stage_v[i] = jnp.zeros((LANES,), jnp.float32)
        return carry

    lax.fori_loop(0, rows, zbody, 0)
    pltpu.sync_copy(stage_v, deg_sh.at[pl.ds(sid * rows, rows)])
    plsc.subcore_barrier()

    def outer(bi, carry):
        pltpu.sync_copy(dst_hbm.at[wid, pl.ds(bi * IB, IB)], idx_v)
        descs = [
            pltpu.async_copy(ones_v, deg_sh.at[idx_v.at[j]], ssem, add=True)
            for j in range(IB)
        ]
        for d in descs:
            d.wait()
        return carry

    lax.fori_loop(0, n_b, outer, 0)
    plsc.subcore_barrier()
    pltpu.sync_copy(deg_sh.at[pl.ds(sid * rows, rows)], stage_v)

    @pl.when(cid == 0)
    def _():
        pltpu.sync_copy(stage_v, out_a.at[pl.ds(sid * rows, rows)])

    @pl.when(cid == 1)
    def _():
        pltpu.sync_copy(stage_v, out_b.at[pl.ds(sid * rows, rows)])


def _zero_acc(zbuf_v, acc_sh, base, pieces):
    def zbody(i, carry):
        zbuf_v[i] = jnp.zeros((LANES,), jnp.float32)
        return carry

    lax.fori_loop(0, CHUNK, zbody, 0)
    for o, ln in pieces:
        pltpu.sync_copy(zbuf_v.at[pl.ds(0, ln)],
                        acc_sh.at[pl.ds(base + o, ln)])


def _edge_loop(src_hbm, dst_hbm, sidx_v, didx_v, rows_v, table_sh, acc_sh,
               gsems, ssem, wid):
    n_b = src_hbm.shape[1] // IB

    def outer(bi, carry):
        i0 = pltpu.async_copy(src_hbm.at[wid, pl.ds(bi * IB, IB)], sidx_v,
                              ssem)
        i1 = pltpu.async_copy(dst_hbm.at[wid, pl.ds(bi * IB, IB)], didx_v,
                              ssem)
        i0.wait()
        i1.wait()
        gd = [
            pltpu.async_copy(table_sh.at[sidx_v.at[j]],
                             rows_v.at[pl.ds(j * CHUNK, CHUNK)], gsems.at[j])
            for j in range(IB)
        ]
        sd = []
        for j in range(IB):
            gd[j].wait()
            sd.append(
                pltpu.async_copy(rows_v.at[pl.ds(j * CHUNK, CHUNK)],
                                 acc_sh.at[didx_v.at[j]], ssem, add=True))
        for d in sd:
            d.wait()
        return carry

    lax.fori_loop(0, n_b, outer, 0)


def _acc_writeback(rows_v, acc_sh, out_a, out_b, base, pieces, cid):
    for k, (o, ln) in enumerate(pieces):
        pltpu.sync_copy(acc_sh.at[pl.ds(base + o, ln)],
                        rows_v.at[pl.ds(k * CHUNK, ln)])

        @pl.when(cid == 0)
        def _(k=k, o=o, ln=ln):
            pltpu.sync_copy(rows_v.at[pl.ds(k * CHUNK, ln)],
                            out_a.at[pl.ds(base + o, ln)])

        @pl.when(cid == 1)
        def _(k=k, o=o, ln=ln):
            pltpu.sync_copy(rows_v.at[pl.ds(k * CHUNK, ln)],
                            out_b.at[pl.ds(base + o, ln)])


def _edge1_body(src_hbm, dst_hbm, da_hbm, db_hbm, h_hbm,
                pa_out, pb_out, hn1_out, dinv_out,
                sidx_v, didx_v, rows_v, zbuf_v, table_sh, acc_sh,
                gsems, ssem):
    n_pad = acc_sh.shape[0]
    rows = n_pad // NS
    cid = lax.axis_index("c")
    sid = lax.axis_index("s")
    wid = sid * NC + cid
    base = sid * rows
    pieces = [(o, min(CHUNK, rows - o)) for o in range(0, rows, CHUNK)]

    _zero_acc(zbuf_v, acc_sh, base, pieces)
    for o, ln in pieces:
        c0 = pltpu.async_copy(da_hbm.at[pl.ds(base + o, ln)],
                              rows_v.at[pl.ds(0, ln)], gsems.at[0])
        c1 = pltpu.async_copy(db_hbm.at[pl.ds(base + o, ln)],
                              rows_v.at[pl.ds(CHUNK, ln)], gsems.at[1])
        c2 = pltpu.async_copy(h_hbm.at[pl.ds(base + o, ln)],
                              rows_v.at[pl.ds(2 * CHUNK, ln)], gsems.at[2])
        c0.wait()
        c1.wait()
        c2.wait()

        def body(i, carry):
            d = rows_v[i] + rows_v[CHUNK + i] + 1.0
            y = _rsqrt_newton(d)
            rows_v[3 * CHUNK + i] = y
            rows_v[4 * CHUNK + i] = rows_v[2 * CHUNK + i] * y
            return carry

        lax.fori_loop(0, ln, body, 0)
        pltpu.sync_copy(rows_v.at[pl.ds(4 * CHUNK, ln)],
                        table_sh.at[pl.ds(base + o, ln)])

        @pl.when(cid == 0)
        def _(o=o, ln=ln):
            pltpu.sync_copy(rows_v.at[pl.ds(4 * CHUNK, ln)],
                            hn1_out.at[pl.ds(base + o, ln)])
            pltpu.sync_copy(rows_v.at[pl.ds(3 * CHUNK, ln)],
                            dinv_out.at[pl.ds(base + o, ln)])

    plsc.subcore_barrier()
    _edge_loop(src_hbm, dst_hbm, sidx_v, didx_v, rows_v, table_sh, acc_sh,
               gsems, ssem, wid)
    plsc.subcore_barrier()
    _acc_writeback(rows_v, acc_sh, pa_out, pb_out, base, pieces, cid)


def _edge2_body(src_hbm, dst_hbm, pa_hbm, pb_hbm, hn1_hbm, dinv_hbm, b1_hbm,
                qa_out, qb_out,
                sidx_v, didx_v, rows_v, zbuf_v, b1_v, table_sh, acc_sh,
                gsems, ssem):
    n_pad = acc_sh.shape[0]
    rows = n_pad // NS
    cid = lax.axis_index("c")
    sid = lax.axis_index("s")
    wid = sid * NC + cid
    base = sid * rows
    pieces = [(o, min(CHUNK, rows - o)) for o in range(0, rows, CHUNK)]

    _zero_acc(zbuf_v, acc_sh, base, pieces)
    pltpu.sync_copy(b1_hbm, b1_v)
    b1 = b1_v[...]
    for o, ln in pieces:
        c0 = pltpu.async_copy(pa_hbm.at[pl.ds(base + o, ln)],
                              rows_v.at[pl.ds(0, ln)], gsems.at[0])
        c1 = pltpu.async_copy(pb_hbm.at[pl.ds(base + o, ln)],
                              rows_v.at[pl.ds(CHUNK, ln)], gsems.at[1])
        c2 = pltpu.async_copy(hn1_hbm.at[pl.ds(base + o, ln)],
                              rows_v.at[pl.ds(2 * CHUNK, ln)], gsems.at[2])
        c3 = pltpu.async_copy(dinv_hbm.at[pl.ds(base + o, ln)],
                              rows_v.at[pl.ds(3 * CHUNK, ln)], gsems.at[3])
        c0.wait()
        c1.wait()
        c2.wait()
        c3.wait()

        def body(i, carry):
            s = rows_v[i] + rows_v[CHUNK + i] + rows_v[2 * CHUNK + i]
            y = rows_v[3 * CHUNK + i]
            o1 = jnp.maximum(y * s + b1, 0.0)
            rows_v[4 * CHUNK + i] = y * o1
            return carry

        lax.fori_loop(0, ln, body, 0)
        pltpu.sync_copy(rows_v.at[pl.ds(4 * CHUNK, ln)],
                        table_sh.at[pl.ds(base + o, ln)])

    plsc.subcore_barrier()
    _edge_loop(src_hbm, dst_hbm, sidx_v, didx_v, rows_v, table_sh, acc_sh,
               gsems, ssem, wid)
    plsc.subcore_barrier()
    f = jnp.where(cid == 0, 1.0, 0.0).astype(jnp.float32)
    for k, (o, ln) in enumerate(pieces):
        c0 = pltpu.async_copy(dinv_hbm.at[pl.ds(base + o, ln)],
                              rows_v.at[pl.ds(0, ln)], gsems.at[0])
        c1 = pltpu.async_copy(acc_sh.at[pl.ds(base + o, ln)],
                              rows_v.at[pl.ds(CHUNK, ln)], gsems.at[1])
        c2 = pltpu.async_copy(table_sh.at[pl.ds(base + o, ln)],
                              rows_v.at[pl.ds(2 * CHUNK, ln)], gsems.at[2])
        c0.wait()
        c1.wait()
        c2.wait()

        def qbody(i, carry):
            q = rows_v[i] * (rows_v[CHUNK + i] + f * rows_v[2 * CHUNK + i])
            rows_v[3 * CHUNK + i] = q
            return carry

        lax.fori_loop(0, ln, qbody, 0)

        @pl.when(cid == 0)
        def _(o=o, ln=ln):
            pltpu.sync_copy(rows_v.at[pl.ds(3 * CHUNK, ln)],
                            qa_out.at[pl.ds(base + o, ln)])

        @pl.when(cid == 1)
        def _(o=o, ln=ln):
            pltpu.sync_copy(rows_v.at[pl.ds(3 * CHUNK, ln)],
                            qb_out.at[pl.ds(base + o, ln)])


def _tc0_body(x_ref, w_ref, h_ref):
    h_ref[...] = jnp.dot(x_ref[...], w_ref[...],
                         preferred_element_type=jnp.float32)


def _tcf_body(qa_ref, qb_ref, w2_ref, b2_ref, out_ref):
    t = qa_ref[...] + qb_ref[...]
    out_ref[...] = (
        jnp.dot(t, w2_ref[...], preferred_element_type=jnp.float32)
        + b2_ref[...])


@functools.partial(jax.jit, static_argnames=("n_pad", "cpw"))
def _run(x_p, srcp, dstp, W1, b1v, w2p, b2r, n_pad, cpw):
    f32 = jnp.float32
    mesh = plsc.VectorSubcoreMesh(core_axis_name="c", subcore_axis_name="s")
    rows = n_pad // NS

    sc_params = pltpu.CompilerParams(use_tc_tiling_on_sc=False,
                                     needs_layout_passes=False)
    sds = jax.ShapeDtypeStruct((n_pad, HIDP), f32)
    deg_call = pl.kernel(
        _deg_body,
        out_type=[sds, sds],
        mesh=mesh,
        compiler_params=sc_params,
        scratch_types=[
            pltpu.VMEM((IB, CHUNK), jnp.int32),
            pltpu.VMEM((CHUNK, HIDP), f32),
            pltpu.VMEM((rows, HIDP), f32),
            pltpu.VMEM_SHARED((n_pad, HIDP), f32),
            pltpu.SemaphoreType.DMA,
        ],
    )
    edge_scratch = [
        pltpu.VMEM((IB, CHUNK), jnp.int32),
        pltpu.VMEM((IB, CHUNK), jnp.int32),
        pltpu.VMEM((IB * CHUNK, HIDP), f32),
        pltpu.VMEM((CHUNK, HIDP), f32),
        pltpu.VMEM_SHARED((n_pad, HIDP), f32),
        pltpu.VMEM_SHARED((n_pad, HIDP), f32),
        pltpu.SemaphoreType.DMA((IB,)),
        pltpu.SemaphoreType.DMA,
    ]
    edge1_call = pl.kernel(
        _edge1_body,
        out_type=[sds, sds, sds, sds],
        mesh=mesh,
        compiler_params=sc_params,
        scratch_types=edge_scratch,
    )
    edge2_call = pl.kernel(
        _edge2_body,
        out_type=[sds, sds],
        mesh=mesh,
        compiler_params=sc_params,
        scratch_types=(edge_scratch[:4]
                       + [pltpu.VMEM((LANES,), f32)]
                       + edge_scratch[4:]),
    )

    grid = 8
    blk = n_pad // grid
    row_spec = pl.BlockSpec((blk, HIDP), lambda i: (i, 0))
    one_spec = pl.BlockSpec((1, HIDP), lambda i: (0, 0))

    da, db = deg_call(dstp)
    h_raw = pl.pallas_call(
        _tc0_body,
        grid=(grid,),
        in_specs=[pl.BlockSpec((blk, x_p.shape[1]), lambda i: (i, 0)),
                  pl.BlockSpec((x_p.shape[1], HIDP), lambda i: (0, 0))],
        out_specs=row_spec,
        out_shape=sds,
    )(x_p, W1)
    p1a, p1b, hn1, dinv = edge1_call(srcp, dstp, da, db, h_raw)
    qa, qb = edge2_call(srcp, dstp, p1a, p1b, hn1, dinv, b1v)
    out_ch = w2p.shape[1]
    outf = pl.pallas_call(
        _tcf_body,
        grid=(grid,),
        in_specs=[row_spec, row_spec,
                  pl.BlockSpec((HIDP, out_ch), lambda i: (0, 0)),
                  pl.BlockSpec((1, out_ch), lambda i: (0, 0))],
        out_specs=pl.BlockSpec((blk, out_ch), lambda i: (i, 0)),
        out_shape=jax.ShapeDtypeStruct((n_pad, out_ch), f32),
    )(qa, qb, w2p, b2r)
    return outf


def kernel(x, edge_index, W1, b1, W2, b2):
    f32 = jnp.float32
    n, in_ch = x.shape
    out_ch = W2.shape[1]
    n_edges = edge_index.shape[1]

    block = NW * CHUNK * IB
    cpw = IB * (-(-n_edges // block))
    e_pad = cpw * NW * CHUNK
    n_pad = -(-n // (NS * 8)) * (NS * 8)
    dummy = n

    ei = edge_index.astype(jnp.int32)
    pad = jnp.full((e_pad - n_edges,), dummy, jnp.int32)
    srcp = jnp.concatenate([ei[0], pad]).reshape(NW, cpw, CHUNK)
    dstp = jnp.concatenate([ei[1], pad]).reshape(NW, cpw, CHUNK)
    x_p = jnp.pad(x.astype(f32), ((0, n_pad - n), (0, 0)))
    b1v = b1.astype(f32)
    w2p = W2.astype(f32)
    b2r = b2.reshape(1, out_ch).astype(f32)

    outf = _run(x_p, srcp, dstp, W1.astype(f32), b1v, w2p, b2r,
                n_pad=n_pad, cpw=cpw)
    return outf[:n]
